# baseline (speedup 1.0000x reference)
import functools

import jax
import jax.numpy as jnp
from jax.experimental import pallas as pl
from jax.experimental.pallas import tpu as pltpu

N = 10000
E = 320000
HEADS = 8
HID = 32
OUT = 128

N_PAD = 10240
B = 128
BW = B + 8
E_TOT = E + N
G = (E_TOT + B - 1) // B
E_PAD = G * B

NEG = -1e30


def _leaky(v):
    return jnp.where(v >= 0, v, 0.2 * v)


def _proj_body(x_ref, w_ref, b_ref, o_ref):
    h = jnp.dot(x_ref[...], w_ref[...], preferred_element_type=jnp.float32)
    o_ref[...] = _leaky(h + b_ref[...])


def _node_body(h_ref, w_ref, asrc_ref, adst_ref, h1_ref, as_ref, ad_ref):
    h1 = jnp.dot(h_ref[...], w_ref[...], preferred_element_type=jnp.float32)
    h1_ref[...] = h1
    as_ref[...] = jnp.dot(h1, asrc_ref[...], preferred_element_type=jnp.float32)
    ad_ref[...] = jnp.dot(h1, adst_ref[...], preferred_element_type=jnp.float32)


def _final_body(num_ref, den_ref, r_ref, b_ref, o_ref):
    denf = jnp.dot(den_ref[...], r_ref[...], preferred_element_type=jnp.float32)
    o_ref[...] = _leaky(num_ref[...] / (denf + 1e-16) + b_ref[...])


def _block_onehot(dst_ref, start):
    rel = dst_ref[...] - start
    cols = jax.lax.broadcasted_iota(jnp.int32, (B, BW), 1)
    return rel == cols


def _edge_logits(ase_ref, ad_ref, m_f32, start):
    ad_slice = ad_ref[pl.ds(start, BW), :]
    ad_e = jnp.dot(m_f32, ad_slice, preferred_element_type=jnp.float32)
    return _leaky(ase_ref[...] + ad_e)


def _max_body(ase_ref, dst_ref, ad_ref, starts_ref, mx_ref):
    g = pl.program_id(0)

    @pl.when(g == 0)
    def _():
        mx_ref[...] = jnp.full(mx_ref.shape, NEG, jnp.float32)

    start = starts_ref[g] * 8
    m = _block_onehot(dst_ref, start)
    e = _edge_logits(ase_ref, ad_ref, m.astype(jnp.float32), start)
    heads = e.shape[1]
    rows = [jnp.max(jnp.where(m, e[:, h:h + 1], NEG), axis=0, keepdims=True)
            for h in range(heads)]
    seg = jnp.concatenate(rows, axis=0).T
    mx_ref[pl.ds(start, BW), :] = jnp.maximum(mx_ref[pl.ds(start, BW), :], seg)


def _acc_body(ase_ref, dst_ref, h_ref, ad_ref, mx_ref, r_ref, starts_ref,
              den_ref, num_ref):
    g = pl.program_id(0)

    @pl.when(g == 0)
    def _():
        den_ref[...] = jnp.zeros(den_ref.shape, jnp.float32)
        num_ref[...] = jnp.zeros(num_ref.shape, jnp.float32)

    start = starts_ref[g] * 8
    m = _block_onehot(dst_ref, start)
    mf = m.astype(jnp.float32)
    e = _edge_logits(ase_ref, ad_ref, mf, start)
    mx_slice = mx_ref[pl.ds(start, BW), :]
    mx_e = jnp.dot(mf, mx_slice, preferred_element_type=jnp.float32)
    ex = jnp.exp(e - mx_e)
    exf = jnp.dot(ex, r_ref[...], preferred_element_type=jnp.float32)
    den_b = jnp.dot(mf.T, ex, preferred_element_type=jnp.float32)
    num_b = jnp.dot(mf.T, exf * h_ref[...], preferred_element_type=jnp.float32)
    den_ref[pl.ds(start, BW), :] = den_ref[pl.ds(start, BW), :] + den_b
    num_ref[pl.ds(start, BW), :] = num_ref[pl.ds(start, BW), :] + num_b


def _full(shape):
    return pl.BlockSpec(shape, lambda g: (0, 0))


def _proj(x, w, b):
    return pl.pallas_call(
        _proj_body,
        out_shape=jax.ShapeDtypeStruct((N_PAD, w.shape[1]), jnp.float32),
    )(x, w, b)


def _node(h, w, a_src, a_dst):
    heads = a_src.shape[1]
    d = w.shape[1]
    return pl.pallas_call(
        _node_body,
        out_shape=(
            jax.ShapeDtypeStruct((N_PAD, d), jnp.float32),
            jax.ShapeDtypeStruct((N_PAD, heads), jnp.float32),
            jax.ShapeDtypeStruct((N_PAD, heads), jnp.float32),
        ),
    )(h, w, a_src, a_dst)


def _seg_max(a_se, dst2, a_d, starts, heads):
    return pl.pallas_call(
        _max_body,
        grid=(G,),
        in_specs=[
            pl.BlockSpec((B, heads), lambda g: (g, 0)),
            pl.BlockSpec((B, 1), lambda g: (g, 0)),
            _full((N_PAD, heads)),
            pl.BlockSpec(memory_space=pltpu.SMEM),
        ],
        out_specs=_full((N_PAD, heads)),
        out_shape=jax.ShapeDtypeStruct((N_PAD, heads), jnp.float32),
    )(a_se, dst2, a_d, starts)


def _seg_acc(a_se, dst2, h_se, a_d, mx, r, starts, heads, d):
    return pl.pallas_call(
        _acc_body,
        grid=(G,),
        in_specs=[
            pl.BlockSpec((B, heads), lambda g: (g, 0)),
            pl.BlockSpec((B, 1), lambda g: (g, 0)),
            pl.BlockSpec((B, d), lambda g: (g, 0)),
            _full((N_PAD, heads)),
            _full((N_PAD, heads)),
            _full((heads, d)),
            pl.BlockSpec(memory_space=pltpu.SMEM),
        ],
        out_specs=(
            _full((N_PAD, heads)),
            _full((N_PAD, d)),
        ),
        out_shape=(
            jax.ShapeDtypeStruct((N_PAD, heads), jnp.float32),
            jax.ShapeDtypeStruct((N_PAD, d), jnp.float32),
        ),
    )(a_se, dst2, h_se, a_d, mx, r, starts)


def _final(num, den, r, bias):
    d = num.shape[1]
    return pl.pallas_call(
        _final_body,
        out_shape=jax.ShapeDtypeStruct((N_PAD, d), jnp.float32),
    )(num, den, r, bias)


def _gat(h, src_s, dst2, starts, edge_mask, W, att_src, att_dst, bias, heads, out_ch):
    d = heads * out_ch
    eye = jnp.eye(heads, dtype=jnp.float32)
    a_src = (att_src[:, :, None] * eye[:, None, :]).reshape(d, heads)
    a_dst = (att_dst[:, :, None] * eye[:, None, :]).reshape(d, heads)
    r = jnp.repeat(eye, out_ch, axis=1)
    h1, a_s, a_d = _node(h, W, a_src, a_dst)
    a_se = jnp.where(edge_mask, a_s[src_s], NEG)
    h_se = h1[src_s]
    mx = _seg_max(a_se, dst2, a_d, starts, heads)
    den, num = _seg_acc(a_se, dst2, h_se, a_d, mx, r, starts, heads, d)
    return _final(num, den, r, jnp.broadcast_to(bias, (N_PAD, d)))


@jax.jit
def kernel(x, edge_index, W0, b0, W1, att_src1, att_dst1, bias1,
           W2, att_src2, att_dst2, bias2, W3, att_src3, att_dst3, bias3):
    loop = jnp.arange(N, dtype=edge_index.dtype)
    src = jnp.concatenate([edge_index[0], loop])
    dst = jnp.concatenate([edge_index[1], loop])
    order = jnp.argsort(dst)
    src_s = jnp.concatenate([src[order],
                             jnp.zeros((E_PAD - E_TOT,), jnp.int32)])
    dst_s = jnp.concatenate([dst[order],
                             jnp.full((E_PAD - E_TOT,), N - 1, jnp.int32)])
    starts = jnp.minimum(dst_s.reshape(G, B)[:, 0] // 8, (N_PAD - BW) // 8)
    dst2 = dst_s[:, None]
    edge_mask = (jnp.arange(E_PAD) < E_TOT)[:, None]

    sigma = jnp.linalg.norm(W0, ord=2)
    xp = jnp.zeros((N_PAD, x.shape[1]), jnp.float32).at[:N].set(x)
    h = _proj(xp, W0 / sigma, jnp.broadcast_to(b0, (N_PAD, W0.shape[1])))
    h = _gat(h, src_s, dst2, starts, edge_mask,
             W1, att_src1, att_dst1, bias1, HEADS, HID)
    h = _gat(h, src_s, dst2, starts, edge_mask,
             W2, att_src2, att_dst2, bias2, HEADS, HID)
    h = _gat(h, src_s, dst2, starts, edge_mask,
             W3, att_src3, att_dst3, bias3, 1, OUT)
    return h[:N]


# speedup vs baseline: 1.5386x; 1.5386x over previous
import functools

import jax
import jax.numpy as jnp
from jax.experimental import pallas as pl
from jax.experimental.pallas import tpu as pltpu

N = 10000
E = 320000
HEADS = 8
HID = 32
OUT = 128

N_PAD = 10240
B = 512
BW = B + 8
E_TOT = E + N
G = (E_TOT + B - 1) // B
E_PAD = G * B

NEG = -1e30


def _leaky(v):
    return jnp.where(v >= 0, v, 0.2 * v)


def _proj_body(x_ref, w_ref, b_ref, o_ref):
    h = jnp.dot(x_ref[...], w_ref[...], preferred_element_type=jnp.float32)
    o_ref[...] = _leaky(h + b_ref[...])


def _node_body(h_ref, w_ref, asrc_ref, adst_ref, h1_ref, as_ref, ad_ref):
    h1 = jnp.dot(h_ref[...], w_ref[...], preferred_element_type=jnp.float32)
    h1_ref[...] = h1
    as_ref[...] = jnp.dot(h1, asrc_ref[...], preferred_element_type=jnp.float32)
    ad_ref[...] = jnp.dot(h1, adst_ref[...], preferred_element_type=jnp.float32)


def _final_body(num_ref, den_ref, r_ref, b_ref, o_ref):
    denf = jnp.dot(den_ref[...], r_ref[...], preferred_element_type=jnp.float32)
    o_ref[...] = _leaky(num_ref[...] / (denf + 1e-16) + b_ref[...])


def _block_onehot(dst_ref, start):
    rel = dst_ref[...] - start
    cols = jax.lax.broadcasted_iota(jnp.int32, (B, BW), 1)
    return rel == cols


def _edge_logits(ase_ref, ad_ref, m_f32, start):
    ad_slice = ad_ref[pl.ds(start, BW), :]
    ad_e = jnp.dot(m_f32, ad_slice, preferred_element_type=jnp.float32)
    return _leaky(ase_ref[...] + ad_e)


def _max_body(ase_ref, dst_ref, ad_ref, starts_ref, mx_ref):
    g = pl.program_id(0)

    @pl.when(g == 0)
    def _():
        mx_ref[...] = jnp.full(mx_ref.shape, NEG, jnp.float32)

    start = starts_ref[g] * 8
    m = _block_onehot(dst_ref, start)
    e = _edge_logits(ase_ref, ad_ref, m.astype(jnp.float32), start)
    heads = e.shape[1]
    rows = [jnp.max(jnp.where(m, e[:, h:h + 1], NEG), axis=0, keepdims=True)
            for h in range(heads)]
    seg = jnp.concatenate(rows, axis=0).T
    mx_ref[pl.ds(start, BW), :] = jnp.maximum(mx_ref[pl.ds(start, BW), :], seg)


def _acc_body(ase_ref, dst_ref, h_ref, ad_ref, mx_ref, r_ref, starts_ref,
              den_ref, num_ref):
    g = pl.program_id(0)

    @pl.when(g == 0)
    def _():
        den_ref[...] = jnp.zeros(den_ref.shape, jnp.float32)
        num_ref[...] = jnp.zeros(num_ref.shape, jnp.float32)

    start = starts_ref[g] * 8
    m = _block_onehot(dst_ref, start)
    mf = m.astype(jnp.float32)
    e = _edge_logits(ase_ref, ad_ref, mf, start)
    mx_slice = mx_ref[pl.ds(start, BW), :]
    mx_e = jnp.dot(mf, mx_slice, preferred_element_type=jnp.float32)
    ex = jnp.exp(e - mx_e)
    exf = jnp.dot(ex, r_ref[...], preferred_element_type=jnp.float32)
    den_b = jnp.dot(mf.T, ex, preferred_element_type=jnp.float32)
    num_b = jnp.dot(mf.T, exf * h_ref[...], preferred_element_type=jnp.float32)
    den_ref[pl.ds(start, BW), :] = den_ref[pl.ds(start, BW), :] + den_b
    num_ref[pl.ds(start, BW), :] = num_ref[pl.ds(start, BW), :] + num_b


def _full(shape):
    return pl.BlockSpec(shape, lambda g: (0, 0))


def _proj(x, w, b):
    return pl.pallas_call(
        _proj_body,
        out_shape=jax.ShapeDtypeStruct((N_PAD, w.shape[1]), jnp.float32),
    )(x, w, b)


def _node(h, w, a_src, a_dst):
    heads = a_src.shape[1]
    d = w.shape[1]
    return pl.pallas_call(
        _node_body,
        out_shape=(
            jax.ShapeDtypeStruct((N_PAD, d), jnp.float32),
            jax.ShapeDtypeStruct((N_PAD, heads), jnp.float32),
            jax.ShapeDtypeStruct((N_PAD, heads), jnp.float32),
        ),
    )(h, w, a_src, a_dst)


def _seg_max(a_se, dst2, a_d, starts, heads):
    return pl.pallas_call(
        _max_body,
        grid=(G,),
        in_specs=[
            pl.BlockSpec((B, heads), lambda g: (g, 0)),
            pl.BlockSpec((B, 1), lambda g: (g, 0)),
            _full((N_PAD, heads)),
            pl.BlockSpec(memory_space=pltpu.SMEM),
        ],
        out_specs=_full((N_PAD, heads)),
        out_shape=jax.ShapeDtypeStruct((N_PAD, heads), jnp.float32),
    )(a_se, dst2, a_d, starts)


def _seg_acc(a_se, dst2, h_se, a_d, mx, r, starts, heads, d):
    return pl.pallas_call(
        _acc_body,
        grid=(G,),
        in_specs=[
            pl.BlockSpec((B, heads), lambda g: (g, 0)),
            pl.BlockSpec((B, 1), lambda g: (g, 0)),
            pl.BlockSpec((B, d), lambda g: (g, 0)),
            _full((N_PAD, heads)),
            _full((N_PAD, heads)),
            _full((heads, d)),
            pl.BlockSpec(memory_space=pltpu.SMEM),
        ],
        out_specs=(
            _full((N_PAD, heads)),
            _full((N_PAD, d)),
        ),
        out_shape=(
            jax.ShapeDtypeStruct((N_PAD, heads), jnp.float32),
            jax.ShapeDtypeStruct((N_PAD, d), jnp.float32),
        ),
    )(a_se, dst2, h_se, a_d, mx, r, starts)


def _final(num, den, r, bias):
    d = num.shape[1]
    return pl.pallas_call(
        _final_body,
        out_shape=jax.ShapeDtypeStruct((N_PAD, d), jnp.float32),
    )(num, den, r, bias)


def _gat(h, src_s, dst2, starts, edge_mask, W, att_src, att_dst, bias, heads, out_ch):
    d = heads * out_ch
    eye = jnp.eye(heads, dtype=jnp.float32)
    a_src = (att_src[:, :, None] * eye[:, None, :]).reshape(d, heads)
    a_dst = (att_dst[:, :, None] * eye[:, None, :]).reshape(d, heads)
    r = jnp.repeat(eye, out_ch, axis=1)
    h1, a_s, a_d = _node(h, W, a_src, a_dst)
    a_se = jnp.where(edge_mask, a_s[src_s], NEG)
    h_se = h1[src_s]
    mx = _seg_max(a_se, dst2, a_d, starts, heads)
    den, num = _seg_acc(a_se, dst2, h_se, a_d, mx, r, starts, heads, d)
    return _final(num, den, r, jnp.broadcast_to(bias, (N_PAD, d)))


@jax.jit
def kernel(x, edge_index, W0, b0, W1, att_src1, att_dst1, bias1,
           W2, att_src2, att_dst2, bias2, W3, att_src3, att_dst3, bias3):
    loop = jnp.arange(N, dtype=edge_index.dtype)
    src = jnp.concatenate([edge_index[0], loop])
    dst = jnp.concatenate([edge_index[1], loop])
    order = jnp.argsort(dst)
    src_s = jnp.concatenate([src[order],
                             jnp.zeros((E_PAD - E_TOT,), jnp.int32)])
    dst_s = jnp.concatenate([dst[order],
                             jnp.full((E_PAD - E_TOT,), N - 1, jnp.int32)])
    starts = jnp.minimum(dst_s.reshape(G, B)[:, 0] // 8, (N_PAD - BW) // 8)
    dst2 = dst_s[:, None]
    edge_mask = (jnp.arange(E_PAD) < E_TOT)[:, None]

    sigma = jnp.linalg.norm(W0, ord=2)
    xp = jnp.zeros((N_PAD, x.shape[1]), jnp.float32).at[:N].set(x)
    h = _proj(xp, W0 / sigma, jnp.broadcast_to(b0, (N_PAD, W0.shape[1])))
    h = _gat(h, src_s, dst2, starts, edge_mask,
             W1, att_src1, att_dst1, bias1, HEADS, HID)
    h = _gat(h, src_s, dst2, starts, edge_mask,
             W2, att_src2, att_dst2, bias2, HEADS, HID)
    h = _gat(h, src_s, dst2, starts, edge_mask,
             W3, att_src3, att_dst3, bias3, 1, OUT)
    return h[:N]
